# baseline (speedup 1.0000x reference)
import functools

import jax
import jax.numpy as jnp
from jax import lax
from jax.experimental import pallas as pl
from jax.experimental.pallas import tpu as pltpu
from jax.experimental.pallas import tpu_sc as plsc

T = 26
B = 4096
L = 20
VOCAB = 100000
D = 64

NC = 2
NS = 16
NW = NC * NS

CHUNK = 16
ROWS = CHUNK * L
NG = 4
GSZ = ROWS // NG
CHUNKS_PER_TABLE = B // CHUNK
TOTAL_CHUNKS = T * CHUNKS_PER_TABLE
CPW = TOTAL_CHUNKS // NW
SUPER = 8
NSUPER = CPW // SUPER
SROWS = SUPER * ROWS
SBAGS = SUPER * CHUNK

REPACK_BLK = 2000


HALF = T * VOCAB // 2


def _repack(weights):
    def body(a_ref, o_ref):
        o_ref[:, 0:D] = a_ref[...]
        o_ref[:, D : 2 * D] = a_ref[...]

    return pl.pallas_call(
        body,
        grid=(T * VOCAB // REPACK_BLK,),
        in_specs=[pl.BlockSpec((REPACK_BLK, D), lambda i: (i, 0))],
        out_specs=pl.BlockSpec((REPACK_BLK, 2 * D), lambda i: (i, 0)),
        out_shape=jax.ShapeDtypeStruct((T * VOCAB, 2 * D), jnp.float32),
    )(weights)


def _tec_body(idx_hbm, w_hbm, out_hbm, idxr, idxa, rows, pooled, sg0, sg1):
    wid = lax.axis_index("s") * NC + lax.axis_index("c")
    base = wid * CPW
    sems = (sg0, sg1)

    def adjust_and_fire(k, tbase, kb):
        off = k * ROWS
        for g in range(NG):
            for jj in range(GSZ // 16):
                v = idxr[pl.ds(off + g * GSZ + jj * 16, 16)] + tbase
                idxa[kb, g, pl.ds(jj * 16, 16)] = v
        return [
            pltpu.async_copy(
                w_hbm.at[idxa.at[kb, g]],
                rows.at[kb, pl.ds(g * GSZ, GSZ)],
                sems[kb],
            )
            for g in range(NG)
        ]

    def accumulate(k, kb):
        p0 = k * CHUNK

        def bag_body(bb, _):
            r0 = bb * L
            acc = [rows[kb, r0, pl.ds(cc * 16, 16)] for cc in range(D // 16)]
            for l in range(1, L):
                for cc in range(D // 16):
                    acc[cc] = acc[cc] + rows[kb, r0 + l, pl.ds(cc * 16, 16)]
            for cc in range(D // 16):
                pooled[p0 + bb, pl.ds(cc * 16, 16)] = acc[cc]
            return 0

        lax.fori_loop(0, CHUNK, bag_body, 0, unroll=2)

    def super_body(s, _):
        c0 = base + s * SUPER
        g0 = c0 * CHUNK
        t = c0 // CHUNKS_PER_TABLE
        b0 = g0 - t * B
        tbase = t * VOCAB

        pltpu.sync_copy(idx_hbm.at[pl.ds(c0 * ROWS, SROWS)], idxr)
        cps = adjust_and_fire(0, tbase, 0)
        for k in range(SUPER):
            nxt = adjust_and_fire(k + 1, tbase, (k + 1) % 2) if k + 1 < SUPER else None
            for cp in cps:
                cp.wait()
            accumulate(k, k % 2)
            cps = nxt

        pltpu.sync_copy(pooled, out_hbm.at[pl.ds(b0, SBAGS), pl.ds(t * D, D)])
        return 0

    lax.fori_loop(0, NSUPER, super_body, 0, unroll=1)


def kernel(indices, offsets, weights):
    del offsets
    w2 = _repack(weights)
    mesh = plsc.VectorSubcoreMesh(core_axis_name="c", subcore_axis_name="s")
    k = functools.partial(
        pl.kernel,
        mesh=mesh,
        compiler_params=pltpu.CompilerParams(use_tc_tiling_on_sc=False),
        out_type=jax.ShapeDtypeStruct((B, T * D), jnp.float32),
        scratch_types=[
            pltpu.VMEM((SROWS,), jnp.int32),
            pltpu.VMEM((2, NG, GSZ), jnp.int32),
            pltpu.VMEM((2, ROWS, 2 * D), jnp.float32),
            pltpu.VMEM((SBAGS, D), jnp.float32),
            pltpu.SemaphoreType.DMA,
            pltpu.SemaphoreType.DMA,
        ],
    )(_tec_body)
    return k(indices, w2)


# speedup vs baseline: 1.1440x; 1.1440x over previous
import functools

import jax
import jax.numpy as jnp
from jax import lax
from jax.experimental import pallas as pl
from jax.experimental.pallas import tpu as pltpu
from jax.experimental.pallas import tpu_sc as plsc

T = 26
B = 4096
L = 20
VOCAB = 100000
D = 64

NC = 2
NS = 16
NW = NC * NS

TPG = 2
NGROUP = T // TPG
CHUNK = 32
ROWS = CHUNK * L
NGATHER = ROWS // 128
CHUNKS_PER_TABLE = B // CHUNK
GROUP_CHUNKS = TPG * CHUNKS_PER_TABLE
CPW = GROUP_CHUNKS // NW
SROWS = CPW * ROWS
SBAGS = CPW * CHUNK


def _tec_body(idx_hbm, w_hbm, out_hbm, idxr, idxa, rows, pooled, sg0, sg1):
    wid = lax.axis_index("s") * NC + lax.axis_index("c")
    c0 = wid * CPW
    t = c0 // CHUNKS_PER_TABLE
    b0 = c0 * CHUNK - t * B
    tbase = t * VOCAB
    sems = (sg0, sg1)

    def adjust_and_fire(k, kb):
        off = k * ROWS

        def adj_body(g, _):
            for jj in range(8):
                v = idxr[pl.ds(off + g * 128 + jj * 16, 16)] + tbase
                idxa[kb, g, pl.ds(jj * 16, 16)] = v
            return 0

        lax.fori_loop(0, NGATHER, adj_body, 0, unroll=1)
        return [
            pltpu.async_copy(
                w_hbm.at[idxa.at[kb, g]],
                rows.at[kb, pl.ds(g * 128, 128)],
                sems[kb],
            )
            for g in range(NGATHER)
        ]

    def accumulate(k, kb):
        p0 = k * CHUNK

        def bag_body(bb, _):
            r0 = bb * L
            acc = [rows[kb, r0, pl.ds(cc * 16, 16)] for cc in range(D // 16)]
            for l in range(1, L):
                for cc in range(D // 16):
                    acc[cc] = acc[cc] + rows[kb, r0 + l, pl.ds(cc * 16, 16)]
            for cc in range(D // 16):
                pooled[p0 + bb, pl.ds(cc * 16, 16)] = acc[cc]
            return 0

        lax.fori_loop(0, CHUNK, bag_body, 0, unroll=2)

    pltpu.sync_copy(idx_hbm.at[pl.ds(c0 * ROWS, SROWS)], idxr)
    cps = adjust_and_fire(0, 0)
    for k in range(CPW):
        nxt = adjust_and_fire(k + 1, (k + 1) % 2) if k + 1 < CPW else None
        for cp in cps:
            cp.wait()
        accumulate(k, k % 2)
        cps = nxt

    pltpu.sync_copy(pooled, out_hbm.at[pl.ds(b0, SBAGS), pl.ds(t * D, D)])


def kernel(indices, offsets, weights):
    del offsets
    mesh = plsc.VectorSubcoreMesh(core_axis_name="c", subcore_axis_name="s")
    k = functools.partial(
        pl.kernel,
        mesh=mesh,
        compiler_params=pltpu.CompilerParams(use_tc_tiling_on_sc=False),
        out_type=jax.ShapeDtypeStruct((B, TPG * D), jnp.float32),
        scratch_types=[
            pltpu.VMEM((SROWS,), jnp.int32),
            pltpu.VMEM((2, NGATHER, 128), jnp.int32),
            pltpu.VMEM((2, ROWS, D), jnp.float32),
            pltpu.VMEM((SBAGS, D), jnp.float32),
            pltpu.SemaphoreType.DMA,
            pltpu.SemaphoreType.DMA,
        ],
    )(_tec_body)
    outs = []
    for g in range(NGROUP):
        wslice = lax.slice_in_dim(weights, g * TPG * VOCAB, (g + 1) * TPG * VOCAB)
        islice = lax.slice_in_dim(indices, g * TPG * B * L, (g + 1) * TPG * B * L)
        outs.append(k(islice, wslice))
    return jnp.concatenate(outs, axis=1)


# speedup vs baseline: 1.5288x; 1.3364x over previous
import functools

import jax
import jax.numpy as jnp
from jax import lax
from jax.experimental import pallas as pl
from jax.experimental.pallas import tpu as pltpu
from jax.experimental.pallas import tpu_sc as plsc

T = 26
B = 4096
L = 20
VOCAB = 100000
D = 64

NC = 2
NS = 16
NW = NC * NS

CHUNK = 32
ROWS = CHUNK * L
NGATHER = ROWS // 128
CHUNKS_PER_TABLE = B // CHUNK
TOTAL_CHUNKS = T * CHUNKS_PER_TABLE
CPW = TOTAL_CHUNKS // NW
SUPER = 8
NSUPER = CPW // SUPER
SROWS = SUPER * ROWS
SBAGS = SUPER * CHUNK


def _tec_body(idx_hbm, w_hbm, out_hbm, idxr, idxa, rows, pooled, sg0, sg1):
    wid = lax.axis_index("s") * NC + lax.axis_index("c")
    base = wid * CPW
    sems = (sg0, sg1)

    def adjust_and_fire(k, tbase, kb):
        off = k * ROWS

        def adj_body(g, _):
            for jj in range(8):
                v = idxr[pl.ds(off + g * 128 + jj * 16, 16)] + tbase
                idxa[kb, g, pl.ds(jj * 16, 16)] = v
            return 0

        lax.fori_loop(0, NGATHER, adj_body, 0, unroll=1)
        return [
            pltpu.async_copy(
                w_hbm.at[idxa.at[kb, g]],
                rows.at[kb, pl.ds(g * 128, 128)],
                sems[kb],
            )
            for g in range(NGATHER)
        ]

    def accumulate(k, kb):
        p0 = k * CHUNK

        def bag_body(bb, _):
            r0 = bb * L
            acc = [rows[kb, r0, pl.ds(cc * 16, 16)] for cc in range(D // 16)]
            for l in range(1, L):
                for cc in range(D // 16):
                    acc[cc] = acc[cc] + rows[kb, r0 + l, pl.ds(cc * 16, 16)]
            for cc in range(D // 16):
                pooled[p0 + bb, pl.ds(cc * 16, 16)] = acc[cc]
            return 0

        lax.fori_loop(0, CHUNK, bag_body, 0, unroll=2)

    def super_body(s, _):
        c0 = base + s * SUPER
        g0 = c0 * CHUNK
        t = c0 // CHUNKS_PER_TABLE
        b0 = g0 - t * B
        tbase = t * VOCAB

        pltpu.sync_copy(idx_hbm.at[pl.ds(c0 * ROWS, SROWS)], idxr)
        cps = adjust_and_fire(0, tbase, 0)
        for k in range(SUPER):
            nxt = adjust_and_fire(k + 1, tbase, (k + 1) % 2) if k + 1 < SUPER else None
            for cp in cps:
                cp.wait()
            accumulate(k, k % 2)
            cps = nxt

        pltpu.sync_copy(pooled, out_hbm.at[pl.ds(b0, SBAGS), pl.ds(t * D, D)])
        return 0

    lax.fori_loop(0, NSUPER, super_body, 0, unroll=1)


def kernel(indices, offsets, weights):
    del offsets
    mesh = plsc.VectorSubcoreMesh(core_axis_name="c", subcore_axis_name="s")
    k = functools.partial(
        pl.kernel,
        mesh=mesh,
        compiler_params=pltpu.CompilerParams(use_tc_tiling_on_sc=False),
        out_type=jax.ShapeDtypeStruct((B, T * D), jnp.float32),
        scratch_types=[
            pltpu.VMEM((SROWS,), jnp.int32),
            pltpu.VMEM((2, NGATHER, 128), jnp.int32),
            pltpu.VMEM((2, ROWS, D), jnp.float32),
            pltpu.VMEM((SBAGS, D), jnp.float32),
            pltpu.SemaphoreType.DMA,
            pltpu.SemaphoreType.DMA,
        ],
    )(_tec_body)
    return k(indices, weights)
